# baseline (speedup 1.0000x reference)
import functools

import jax
import jax.numpy as jnp
from jax import lax
from jax.experimental import pallas as pl
from jax.experimental.pallas import tpu as pltpu
from jax.experimental.pallas import tpu_sc as plsc

VOCAB = 100000
EMBED = 64
MAX_POS = 512
B, S = 4096, 200
N = B * S

NC, NS = 2, 16
NW = NC * NS
RPW = N // NW
CHUNK = 800
NCHUNK = RPW // CHUNK
SUB = 100
NSUB = CHUNK // SUB
GROUPS = CHUNK // 16


def _sc_embed_ln(ids2d, word_emb, pos_emb, gamma, beta):
  mesh = plsc.VectorSubcoreMesh(core_axis_name="c", subcore_axis_name="s")

  @functools.partial(
      pl.kernel,
      out_type=jax.ShapeDtypeStruct((N, EMBED), jnp.float32),
      mesh=mesh,
      compiler_params=pltpu.CompilerParams(
          needs_layout_passes=False, use_tc_tiling_on_sc=False),
      scratch_types=[
          pltpu.VMEM((NSUB, SUB), jnp.int32),
          pltpu.VMEM((CHUNK, EMBED), jnp.float32),
          pltpu.VMEM((S, EMBED), jnp.float32),
          pltpu.VMEM((EMBED,), jnp.float32),
          pltpu.VMEM((EMBED,), jnp.float32),
          pltpu.VMEM((EMBED * 16,), jnp.float32),
          pltpu.SemaphoreType.DMA,
      ],
  )
  def k(ids_hbm, wtab_hbm, pos_hbm, g_hbm, b_hbm, out_hbm,
        idx_v, rows_v, pos_v, g_v, b_v, tmp_v, sem):
    wid = lax.axis_index("s") * NC + lax.axis_index("c")

    pltpu.sync_copy(pos_hbm.at[pl.ds(0, S)], pos_v)
    pltpu.sync_copy(g_hbm, g_v)
    pltpu.sync_copy(b_hbm, b_v)

    iota16 = lax.iota(jnp.int32, 16)

    def chunk_body(c, carry):
      base = pl.multiple_of(wid * RPW + c * CHUNK, CHUNK)
      pltpu.sync_copy(ids_hbm.at[pl.ds(pl.multiple_of(base // SUB, NSUB), NSUB)],
                      idx_v)
      cps = [
          pltpu.async_copy(wtab_hbm.at[idx_v.at[q]],
                           rows_v.at[pl.ds(q * SUB, SUB)], sem)
          for q in range(NSUB)
      ]
      for cp in cps:
        cp.wait()

      def group_body(g, carry2):
        ridx = g * 16 + iota16
        prow = lax.rem(ridx, S)
        acc_s = jnp.zeros((16,), jnp.float32)
        acc_q = jnp.zeros((16,), jnp.float32)
        for j in range(EMBED):
          cj = jnp.full((16,), j, jnp.int32)
          w = plsc.load_gather(rows_v, [ridx, cj])
          p = plsc.load_gather(pos_v, [prow, cj])
          x = w + p
          acc_s = acc_s + x
          acc_q = acc_q + x * x
          tmp_v[pl.ds(j * 16, 16)] = x
        mu = acc_s * (1.0 / EMBED)
        var = acc_q * (1.0 / EMBED) - mu * mu
        t = var + 1e-5
        yi = jnp.int32(0x5F3759DF) - lax.shift_right_arithmetic(
            plsc.bitcast(t, jnp.int32), 1)
        y = plsc.bitcast(yi, jnp.float32)
        for _ in range(3):
          y = y * (1.5 - 0.5 * t * y * y)
        for j in range(EMBED):
          cj = jnp.full((16,), j, jnp.int32)
          x = tmp_v[pl.ds(j * 16, 16)]
          gj = plsc.load_gather(g_v, [cj])
          bj = plsc.load_gather(b_v, [cj])
          out = (x - mu) * y * gj + bj
          plsc.store_scatter(rows_v, [ridx, cj], out)
        return carry2

      lax.fori_loop(0, GROUPS, group_body, 0)

      pltpu.sync_copy(rows_v, out_hbm.at[pl.ds(base, CHUNK)])
      return carry

    lax.fori_loop(0, NCHUNK, chunk_body, 0)

  return k(ids2d, word_emb, pos_emb, gamma, beta)


def kernel(input_ids, word_emb, pos_emb, gamma, beta):
  ids2d = input_ids.astype(jnp.int32).reshape(N // SUB, SUB)
  out = _sc_embed_ln(ids2d, word_emb, pos_emb, gamma, beta)
  return out.reshape(B, S, EMBED)
